# Initial kernel scaffold; baseline (speedup 1.0000x reference)
import jax
import jax.numpy as jnp
from jax.experimental import pallas as pl


def kernel(hidden_states, router_w, gate_w, up_w, down_w, shared_gate_w, shared_up_w, shared_down_w):
    raise NotImplementedError("write your pallas kernel here")


# speedup vs baseline: 1.3315x; 1.3315x over previous
import functools

import jax
import jax.numpy as jnp
from jax import lax
from jax.experimental import pallas as pl
from jax.experimental.pallas import tpu as pltpu
from jax.experimental.pallas import tpu_sc as plsc

N = 4096
H = 768
F = 2048
E = 8
K = 2
TILE = 256
A = N * K
G_ROUTED = A // TILE + E
M_ROUTED = G_ROUTED * TILE
M = M_ROUTED + N
G = G_ROUTED + N // TILE
NW = 32
TPW = N // NW
CH_S = 64
CH_C = 32


def _router_body(flat_ref, rw_ref, logits_ref, p1_ref, p2_ref, w1_ref,
                 w2_ref, te_ref, aux_ref, z_ref):
    x = flat_ref[...]
    rw = rw_ref[...]
    logits = lax.dot_general(x, rw, (((1,), (1,)), ((), ())),
                             preferred_element_type=jnp.float32)
    logits_ref[...] = logits
    m = jnp.max(logits, axis=1, keepdims=True)
    ex = jnp.exp(logits - m)
    sex = jnp.sum(ex, axis=1, keepdims=True)
    probs = ex / sex
    iota = lax.broadcasted_iota(jnp.int32, (N, E), 1)
    m1 = jnp.max(probs, axis=1, keepdims=True)
    e1 = jnp.min(jnp.where(probs == m1, iota, E), axis=1, keepdims=True)
    masked = jnp.where(iota == e1, -1.0, probs)
    m2 = jnp.max(masked, axis=1, keepdims=True)
    e2 = jnp.min(jnp.where(masked == m2, iota, E), axis=1, keepdims=True)
    wsum = m1 + m2
    w1_ref[...] = m1 / wsum
    w2_ref[...] = m2 / wsum

    c1 = jnp.where(iota == e1, 1.0, 0.0)
    c2 = jnp.where(iota == e2, 1.0, 0.0)
    cmat = c1 + c2
    counts = jnp.sum(cmat, axis=0, keepdims=True)

    tpe = counts / float(K * N)
    rppe = jnp.mean(probs, axis=0, keepdims=True)
    aux_ref[...] = (float(E) * jnp.sum(tpe * rppe)).reshape(1, 1)
    lse = m + jnp.log(sex)
    z_ref[...] = jnp.mean(lse * lse).reshape(1, 1)

    pc = jnp.ceil(counts / float(TILE)) * float(TILE)
    su_iota_r = lax.broadcasted_iota(jnp.int32, (E, E), 0)
    su_iota_c = lax.broadcasted_iota(jnp.int32, (E, E), 1)
    su = jnp.where(su_iota_r < su_iota_c, 1.0, 0.0)
    off = lax.dot_general(pc, su, (((1,), (0,)), ((), ())),
                          preferred_element_type=jnp.float32)

    g_row = lax.broadcasted_iota(jnp.int32, (G, E), 0) * TILE
    offi = off.astype(jnp.int32)
    cnt = jnp.sum(jnp.where(g_row >= offi, 1, 0), axis=1, keepdims=True) - 1
    te = jnp.clip(cnt, 0, E - 1)
    gidx = lax.broadcasted_iota(jnp.int32, (G, 1), 0)
    te_ref[...] = jnp.where(gidx >= G_ROUTED, E, te)

    chunk = 512
    for i in range(N // chunk):
        r0 = i * chunk
        ri = lax.broadcasted_iota(jnp.int32, (chunk, N), 0) + r0
        ci = lax.broadcasted_iota(jnp.int32, (chunk, N), 1)
        tri = jnp.where(ri > ci, 1.0, 0.0)
        rank = lax.dot_general(tri, cmat, (((1,), (0,)), ((), ())),
                               preferred_element_type=jnp.float32)
        pos = off + rank
        e1c = e1[r0:r0 + chunk, :]
        e2c = e2[r0:r0 + chunk, :]
        eio = lax.broadcasted_iota(jnp.int32, (chunk, E), 1)
        p1c = jnp.sum(jnp.where(eio == e1c, pos, 0.0), axis=1, keepdims=True)
        p2c = jnp.sum(jnp.where(eio == e2c, pos, 0.0), axis=1, keepdims=True)
        p1_ref[r0:r0 + chunk, :] = p1c.astype(jnp.int32)
        p2_ref[r0:r0 + chunk, :] = p2c.astype(jnp.int32)


def _router_call(flat, router_w):
    return pl.pallas_call(
        _router_body,
        out_shape=[
            jax.ShapeDtypeStruct((N, E), jnp.float32),
            jax.ShapeDtypeStruct((N, 1), jnp.int32),
            jax.ShapeDtypeStruct((N, 1), jnp.int32),
            jax.ShapeDtypeStruct((N, 1), jnp.float32),
            jax.ShapeDtypeStruct((N, 1), jnp.float32),
            jax.ShapeDtypeStruct((G, 1), jnp.int32),
            jax.ShapeDtypeStruct((1, 1), jnp.float32),
            jax.ShapeDtypeStruct((1, 1), jnp.float32),
        ],
    )(flat, router_w)


def _scatter_body(flat, p1, p2, w1, w2, xs, ws,
                  rows_v, i1_v, i2_v, v1_v, v2_v, one_v, sem, sem2):
    cid = lax.axis_index("c")
    sid = lax.axis_index("s")
    wid = sid * 2 + cid
    for i in range(CH_S // 16):
        one_v[pl.ds(i * 16, 16)] = jnp.full((16,), 1.0, jnp.float32)
    for c in range(TPW // CH_S):
        b = wid * TPW + c * CH_S
        pltpu.sync_copy(flat.at[pl.ds(b, CH_S)], rows_v)
        pltpu.sync_copy(p1.at[pl.ds(b, CH_S)], i1_v)
        pltpu.sync_copy(p2.at[pl.ds(b, CH_S)], i2_v)
        pltpu.sync_copy(w1.at[pl.ds(b, CH_S)], v1_v)
        pltpu.sync_copy(w2.at[pl.ds(b, CH_S)], v2_v)
        pltpu.async_copy(rows_v, xs.at[i1_v], sem).wait()
        pltpu.async_copy(rows_v, xs.at[i2_v], sem).wait()
        pltpu.async_copy(v1_v, ws.at[i1_v], sem2).wait()
        pltpu.async_copy(v2_v, ws.at[i2_v], sem2).wait()
        pltpu.sync_copy(rows_v, xs.at[pl.ds(M_ROUTED + b, CH_S)])
        pltpu.sync_copy(one_v, ws.at[pl.ds(M_ROUTED + b, CH_S)])


def _scatter_call(flat, p1, p2, w1, w2):
    f = functools.partial(
        pl.kernel,
        out_type=[
            jax.ShapeDtypeStruct((M, H), jnp.float32),
            jax.ShapeDtypeStruct((M,), jnp.float32),
        ],
        mesh=plsc.VectorSubcoreMesh(core_axis_name="c", subcore_axis_name="s"),
        scratch_types=[
            pltpu.VMEM((CH_S, H), jnp.float32),
            pltpu.VMEM((CH_S,), jnp.int32),
            pltpu.VMEM((CH_S,), jnp.int32),
            pltpu.VMEM((CH_S,), jnp.float32),
            pltpu.VMEM((CH_S,), jnp.float32),
            pltpu.VMEM((CH_S,), jnp.float32),
            pltpu.SemaphoreType.DMA,
            pltpu.SemaphoreType.DMA,
        ],
    )(_scatter_body)
    return f(flat, p1, p2, w1, w2)


def _gemm_body(te_ref, x_ref, gw_ref, uw_ref, dw_ref, ws_ref, y_ref):
    x = x_ref[...]
    g = lax.dot_general(x, gw_ref[0], (((1,), (1,)), ((), ())),
                        preferred_element_type=jnp.float32)
    u = lax.dot_general(x, uw_ref[0], (((1,), (1,)), ((), ())),
                        preferred_element_type=jnp.float32)
    h = g * lax.logistic(g) * u
    y = lax.dot_general(h, dw_ref[0], (((1,), (1,)), ((), ())),
                        preferred_element_type=jnp.float32)
    y_ref[...] = y * ws_ref[...]


def _gemm_call(te, xs, gws, uws, dws, ws):
    grid_spec = pltpu.PrefetchScalarGridSpec(
        num_scalar_prefetch=1,
        grid=(G,),
        in_specs=[
            pl.BlockSpec((TILE, H), lambda g, te: (g, 0)),
            pl.BlockSpec((1, F, H), lambda g, te: (te[g], 0, 0)),
            pl.BlockSpec((1, F, H), lambda g, te: (te[g], 0, 0)),
            pl.BlockSpec((1, H, F), lambda g, te: (te[g], 0, 0)),
            pl.BlockSpec((TILE, 1), lambda g, te: (g, 0)),
        ],
        out_specs=pl.BlockSpec((TILE, H), lambda g, te: (g, 0)),
    )
    return pl.pallas_call(
        _gemm_body,
        grid_spec=grid_spec,
        out_shape=jax.ShapeDtypeStruct((M, H), jnp.float32),
        compiler_params=pltpu.CompilerParams(
            dimension_semantics=("arbitrary",)),
    )(te, xs, gws, uws, dws, ws)


def _combine_body(ys, p1, p2, routed, i1_v, i2_v, r1_v, r2_v, r3_v, out_v, sem):
    cid = lax.axis_index("c")
    sid = lax.axis_index("s")
    wid = sid * 2 + cid
    for c in range(TPW // CH_C):
        b = wid * TPW + c * CH_C
        pltpu.sync_copy(p1.at[pl.ds(b, CH_C)], i1_v)
        pltpu.sync_copy(p2.at[pl.ds(b, CH_C)], i2_v)
        pltpu.async_copy(ys.at[i1_v], r1_v, sem).wait()
        pltpu.async_copy(ys.at[i2_v], r2_v, sem).wait()
        pltpu.sync_copy(ys.at[pl.ds(M_ROUTED + b, CH_C)], r3_v)

        def body(t, carry):
            for j in range(H // 16):
                s = pl.ds(j * 16, 16)
                out_v[t, s] = r1_v[t, s] + r2_v[t, s] + r3_v[t, s]
            return carry

        lax.fori_loop(0, CH_C, body, 0)
        pltpu.sync_copy(out_v, routed.at[pl.ds(b, CH_C)])


def _combine_call(ys, p1, p2):
    f = functools.partial(
        pl.kernel,
        out_type=jax.ShapeDtypeStruct((N, H), jnp.float32),
        mesh=plsc.VectorSubcoreMesh(core_axis_name="c", subcore_axis_name="s"),
        scratch_types=[
            pltpu.VMEM((CH_C,), jnp.int32),
            pltpu.VMEM((CH_C,), jnp.int32),
            pltpu.VMEM((CH_C, H), jnp.float32),
            pltpu.VMEM((CH_C, H), jnp.float32),
            pltpu.VMEM((CH_C, H), jnp.float32),
            pltpu.VMEM((CH_C, H), jnp.float32),
            pltpu.SemaphoreType.DMA,
        ],
    )(_combine_body)
    return f(ys, p1, p2)


def kernel(hidden_states, router_w, gate_w, up_w, down_w,
           shared_gate_w, shared_up_w, shared_down_w):
    b, s, h = hidden_states.shape
    flat = hidden_states.reshape(N, H)
    logits, p1, p2, w1, w2, te, aux, z = _router_call(flat, router_w)
    p1 = p1.reshape(N)
    p2 = p2.reshape(N)
    w1 = w1.reshape(N)
    w2 = w2.reshape(N)
    te = te.reshape(G)

    gws = jnp.concatenate([gate_w, shared_gate_w], axis=0)
    uws = jnp.concatenate([up_w, shared_up_w], axis=0)
    dws = jnp.concatenate([down_w, shared_down_w], axis=0)

    xs, ws = _scatter_call(flat, p1, p2, w1, w2)
    ys = _gemm_call(te, xs, gws, uws, dws, ws.reshape(M, 1))
    routed = _combine_call(ys, p1, p2)

    return (routed.reshape(b, s, h), aux.reshape(()), z.reshape(()),
            logits.reshape(b, s, E))


# speedup vs baseline: 1.4940x; 1.1221x over previous
import functools

import jax
import jax.numpy as jnp
from jax import lax
from jax.experimental import pallas as pl
from jax.experimental.pallas import tpu as pltpu
from jax.experimental.pallas import tpu_sc as plsc

N = 4096
H = 768
F = 2048
E = 8
K = 2
TILE = 256
A = N * K
G_ROUTED = A // TILE + E
M_ROUTED = G_ROUTED * TILE
M = M_ROUTED + N
G = G_ROUTED + N // TILE
NW = 32
TPW = N // NW
CH_S = 64
CH_C = 16


def _router_body(flat_ref, rw_ref, logits_ref, p1_ref, p2_ref, w1_ref,
                 w2_ref, te_ref, aux_ref, z_ref):
    x = flat_ref[...]
    rw = rw_ref[...]
    logits = lax.dot_general(x, rw, (((1,), (1,)), ((), ())),
                             preferred_element_type=jnp.float32)
    logits_ref[...] = logits
    m = jnp.max(logits, axis=1, keepdims=True)
    ex = jnp.exp(logits - m)
    sex = jnp.sum(ex, axis=1, keepdims=True)
    probs = ex / sex
    iota = lax.broadcasted_iota(jnp.int32, (N, E), 1)
    m1 = jnp.max(probs, axis=1, keepdims=True)
    e1 = jnp.min(jnp.where(probs == m1, iota, E), axis=1, keepdims=True)
    masked = jnp.where(iota == e1, -1.0, probs)
    m2 = jnp.max(masked, axis=1, keepdims=True)
    e2 = jnp.min(jnp.where(masked == m2, iota, E), axis=1, keepdims=True)
    wsum = m1 + m2
    ones16 = jnp.ones((1, 16), jnp.float32)
    w1_ref[...] = (m1 / wsum) * ones16
    w2_ref[...] = (m2 / wsum) * ones16

    c1 = jnp.where(iota == e1, 1.0, 0.0)
    c2 = jnp.where(iota == e2, 1.0, 0.0)
    cmat = c1 + c2
    counts = jnp.sum(cmat, axis=0, keepdims=True)

    tpe = counts / float(K * N)
    rppe = jnp.mean(probs, axis=0, keepdims=True)
    aux_ref[...] = (float(E) * jnp.sum(tpe * rppe)).reshape(1, 1)
    lse = m + jnp.log(sex)
    z_ref[...] = jnp.mean(lse * lse).reshape(1, 1)

    pc = jnp.ceil(counts / float(TILE)) * float(TILE)
    su_iota_r = lax.broadcasted_iota(jnp.int32, (E, E), 0)
    su_iota_c = lax.broadcasted_iota(jnp.int32, (E, E), 1)
    su = jnp.where(su_iota_r < su_iota_c, 1.0, 0.0)
    off = lax.dot_general(pc, su, (((1,), (0,)), ((), ())),
                          preferred_element_type=jnp.float32)

    g_row = lax.broadcasted_iota(jnp.int32, (G, E), 0) * TILE
    offi = off.astype(jnp.int32)
    cnt = jnp.sum(jnp.where(g_row >= offi, 1, 0), axis=1, keepdims=True) - 1
    te = jnp.clip(cnt, 0, E - 1)
    gidx = lax.broadcasted_iota(jnp.int32, (G, 1), 0)
    te_ref[...] = jnp.where(gidx >= G_ROUTED, E, te)

    chunk = 512
    for i in range(N // chunk):
        r0 = i * chunk
        ri = lax.broadcasted_iota(jnp.int32, (chunk, N), 0) + r0
        ci = lax.broadcasted_iota(jnp.int32, (chunk, N), 1)
        tri = jnp.where(ri > ci, 1.0, 0.0)
        rank = lax.dot_general(tri, cmat, (((1,), (0,)), ((), ())),
                               preferred_element_type=jnp.float32)
        pos = off + rank
        e1c = e1[r0:r0 + chunk, :]
        e2c = e2[r0:r0 + chunk, :]
        eio = lax.broadcasted_iota(jnp.int32, (chunk, E), 1)
        p1c = jnp.sum(jnp.where(eio == e1c, pos, 0.0), axis=1, keepdims=True)
        p2c = jnp.sum(jnp.where(eio == e2c, pos, 0.0), axis=1, keepdims=True)
        p1_ref[r0:r0 + chunk, :] = p1c.astype(jnp.int32)
        p2_ref[r0:r0 + chunk, :] = p2c.astype(jnp.int32)


def _router_call(flat, router_w):
    return pl.pallas_call(
        _router_body,
        out_shape=[
            jax.ShapeDtypeStruct((N, E), jnp.float32),
            jax.ShapeDtypeStruct((N, 1), jnp.int32),
            jax.ShapeDtypeStruct((N, 1), jnp.int32),
            jax.ShapeDtypeStruct((N, 16), jnp.float32),
            jax.ShapeDtypeStruct((N, 16), jnp.float32),
            jax.ShapeDtypeStruct((G, 1), jnp.int32),
            jax.ShapeDtypeStruct((1, 1), jnp.float32),
            jax.ShapeDtypeStruct((1, 1), jnp.float32),
        ],
    )(flat, router_w)


def _scatter_body(flat, p1, p2, xs, rows0_v, rows1_v, i1a_v, i2a_v,
                  i1b_v, i2b_v, semin, semout):
    cid = lax.axis_index("c")
    sid = lax.axis_index("s")
    wid = sid * 2 + cid
    b0 = wid * TPW
    b1 = wid * TPW + CH_S
    in0 = pltpu.async_copy(flat.at[pl.ds(b0, CH_S)], rows0_v, semin)
    in1 = pltpu.async_copy(flat.at[pl.ds(b1, CH_S)], rows1_v, semin)
    pltpu.sync_copy(p1.at[pl.ds(b0, CH_S)], i1a_v)
    pltpu.sync_copy(p2.at[pl.ds(b0, CH_S)], i2a_v)
    pltpu.sync_copy(p1.at[pl.ds(b1, CH_S)], i1b_v)
    pltpu.sync_copy(p2.at[pl.ds(b1, CH_S)], i2b_v)
    in0.wait()
    o1 = pltpu.async_copy(rows0_v, xs.at[i1a_v], semout)
    o2 = pltpu.async_copy(rows0_v, xs.at[i2a_v], semout)
    in1.wait()
    o3 = pltpu.async_copy(rows1_v, xs.at[i1b_v], semout)
    o4 = pltpu.async_copy(rows1_v, xs.at[i2b_v], semout)
    o1.wait()
    o2.wait()
    o3.wait()
    o4.wait()


def _scatter_call(flat, p1, p2):
    f = functools.partial(
        pl.kernel,
        out_type=jax.ShapeDtypeStruct((M_ROUTED, H), jnp.float32),
        mesh=plsc.VectorSubcoreMesh(core_axis_name="c", subcore_axis_name="s"),
        scratch_types=[
            pltpu.VMEM((CH_S, H), jnp.float32),
            pltpu.VMEM((CH_S, H), jnp.float32),
            pltpu.VMEM((CH_S,), jnp.int32),
            pltpu.VMEM((CH_S,), jnp.int32),
            pltpu.VMEM((CH_S,), jnp.int32),
            pltpu.VMEM((CH_S,), jnp.int32),
            pltpu.SemaphoreType.DMA,
            pltpu.SemaphoreType.DMA,
        ],
    )(_scatter_body)
    return f(flat, p1, p2)


def _gemm_body(te_ref, xs_ref, fl_ref, gw_ref, uw_ref, dw_ref, y_ref):
    g_id = pl.program_id(0)
    x = jnp.where(g_id < G_ROUTED, xs_ref[...], fl_ref[...])
    g = lax.dot_general(x, gw_ref[0], (((1,), (1,)), ((), ())),
                        preferred_element_type=jnp.float32)
    u = lax.dot_general(x, uw_ref[0], (((1,), (1,)), ((), ())),
                        preferred_element_type=jnp.float32)
    h = g * lax.logistic(g) * u
    y = lax.dot_general(h, dw_ref[0], (((1,), (1,)), ((), ())),
                        preferred_element_type=jnp.float32)
    y_ref[...] = y


def _gemm_call(te, xs, flat, gws, uws, dws):
    grid_spec = pltpu.PrefetchScalarGridSpec(
        num_scalar_prefetch=1,
        grid=(G,),
        in_specs=[
            pl.BlockSpec((TILE, H),
                         lambda g, te: (jnp.minimum(g, G_ROUTED - 1), 0)),
            pl.BlockSpec((TILE, H),
                         lambda g, te: (jnp.clip(g - G_ROUTED, 0,
                                                 N // TILE - 1), 0)),
            pl.BlockSpec((1, F, H), lambda g, te: (te[g], 0, 0)),
            pl.BlockSpec((1, F, H), lambda g, te: (te[g], 0, 0)),
            pl.BlockSpec((1, H, F), lambda g, te: (te[g], 0, 0)),
        ],
        out_specs=pl.BlockSpec((TILE, H), lambda g, te: (g, 0)),
    )
    return pl.pallas_call(
        _gemm_body,
        grid_spec=grid_spec,
        out_shape=jax.ShapeDtypeStruct((M, H), jnp.float32),
        compiler_params=pltpu.CompilerParams(
            dimension_semantics=("arbitrary",)),
    )(te, xs, flat, gws, uws, dws)


def _combine_body(ys, p1, p2, w1, w2, routed, i1_v, i2_v, v1_v, v2_v,
                  r1a, r2a, r3a, r1b, r2b, r3b, out, semg, semo):
    cid = lax.axis_index("c")
    sid = lax.axis_index("s")
    wid = sid * 2 + cid
    base = wid * TPW
    pltpu.sync_copy(p1.at[pl.ds(base, TPW)], i1_v)
    pltpu.sync_copy(p2.at[pl.ds(base, TPW)], i2_v)
    pltpu.sync_copy(w1.at[pl.ds(base, TPW)], v1_v)
    pltpu.sync_copy(w2.at[pl.ds(base, TPW)], v2_v)
    nch = TPW // CH_C
    bufs = [(r1a, r2a, r3a), (r1b, r2b, r3b)]

    def issue(c):
        r1, r2, r3 = bufs[c % 2]
        i1c = i1_v[pl.ds(c * CH_C, CH_C)]
        i2c = i2_v[pl.ds(c * CH_C, CH_C)]
        h1 = pltpu.async_copy(ys.at[i1c], r1, semg)
        h2 = pltpu.async_copy(ys.at[i2c], r2, semg)
        h3 = pltpu.async_copy(
            ys.at[pl.ds(M_ROUTED + base + c * CH_C, CH_C)], r3, semg)
        return h1, h2, h3

    hs = issue(0)
    out_h = None
    for c in range(nch):
        r1, r2, r3 = bufs[c % 2]
        nxt = issue(c + 1) if c + 1 < nch else None
        for h in hs:
            h.wait()
        if out_h is not None:
            out_h.wait()

        def body(t, carry):
            tok = c * CH_C + t
            w1b = v1_v[tok, 0:16]
            w2b = v2_v[tok, 0:16]
            for j in range(H // 16):
                s = pl.ds(j * 16, 16)
                out[t, s] = r1[t, s] * w1b + r2[t, s] * w2b + r3[t, s]
            return carry

        lax.fori_loop(0, CH_C, body, 0)
        out_h = pltpu.async_copy(
            out, routed.at[pl.ds(base + c * CH_C, CH_C)], semo)
        hs = nxt
    out_h.wait()


def _combine_call(ys, p1, p2, w1, w2):
    rowbuf = pltpu.VMEM((CH_C, H), jnp.float32)
    f = functools.partial(
        pl.kernel,
        out_type=jax.ShapeDtypeStruct((N, H), jnp.float32),
        mesh=plsc.VectorSubcoreMesh(core_axis_name="c", subcore_axis_name="s"),
        scratch_types=[
            pltpu.VMEM((TPW,), jnp.int32),
            pltpu.VMEM((TPW,), jnp.int32),
            pltpu.VMEM((TPW, 16), jnp.float32),
            pltpu.VMEM((TPW, 16), jnp.float32),
            rowbuf, rowbuf, rowbuf,
            rowbuf, rowbuf, rowbuf,
            rowbuf,
            pltpu.SemaphoreType.DMA,
            pltpu.SemaphoreType.DMA,
        ],
    )(_combine_body)
    return f(ys, p1, p2, w1, w2)


def kernel(hidden_states, router_w, gate_w, up_w, down_w,
           shared_gate_w, shared_up_w, shared_down_w):
    b, s, h = hidden_states.shape
    flat = hidden_states.reshape(N, H)
    logits, p1, p2, w1, w2, te, aux, z = _router_call(flat, router_w)
    p1 = p1.reshape(N)
    p2 = p2.reshape(N)
    te = te.reshape(G)

    gws = jnp.concatenate([gate_w, shared_gate_w], axis=0)
    uws = jnp.concatenate([up_w, shared_up_w], axis=0)
    dws = jnp.concatenate([down_w, shared_down_w], axis=0)

    xs = _scatter_call(flat, p1, p2)
    ys = _gemm_call(te, xs, flat, gws, uws, dws)
    routed = _combine_call(ys, p1, p2, w1, w2)

    return (routed.reshape(b, s, h), aux.reshape(()), z.reshape(()),
            logits.reshape(b, s, E))
